# baseline (speedup 1.0000x reference)
import numpy as np

import jax
import jax.numpy as jnp
from jax import lax
from jax.experimental import pallas as pl
from jax.experimental.pallas import tpu as pltpu
from jax.experimental.pallas import tpu_sc as plsc

NC = 2
NS = 16
NW = NC * NS

E = 320000
D = 128
DW = D // 2
N_NODES = 10000

EPW = E // NW
C = 40
NCHUNK = EPW // C
NSLOT = 4

_PERM = np.array([32 * (m // 32) + (m % 32 % 2) * 16 + (m % 32) // 2
                  for m in range(D)], dtype=np.int32)
_HIMASK = np.int32(-65536)


def _body(xq_hbm, ei_hbm, out_hbm, x_sh, idx0_v, idx1_v, q0, q1, rf,
          sem0, sem1, sem2, sem3):
    sems = [sem0, sem1, sem2, sem3]
    sid = lax.axis_index("s")
    wid = sid * NC + lax.axis_index("c")
    base = pl.multiple_of(wid * EPW, EPW)

    @pl.when(sid < 5)
    def _stage():
        r0 = pl.multiple_of(sid * 2000, 2000)
        pltpu.sync_copy(xq_hbm.at[pl.ds(r0, 2000)], x_sh.at[pl.ds(r0, 2000)])

    pltpu.sync_copy(ei_hbm.at[pl.ds(base, EPW)], idx0_v)
    pltpu.sync_copy(ei_hbm.at[pl.ds(E + base, EPW)], idx1_v)
    plsc.subcore_barrier()

    def _wait_gather(slot):
        pltpu.make_async_copy(
            xq_hbm.at[pl.ds(0, C)], q0.at[slot], sems[slot]).wait()
        pltpu.make_async_copy(
            xq_hbm.at[pl.ds(0, C)], q1.at[slot], sems[slot]).wait()

    def _wait_write(slot):
        pltpu.make_async_copy(
            rf.at[slot], out_hbm.at[pl.ds(0, C)], sems[slot]).wait()

    def group(t, _):
        for u in range(NSLOT):
            j = t * NSLOT + u

            @pl.when(jnp.logical_and(j >= NSLOT, j - NSLOT < NCHUNK))
            def _a():
                _wait_write(u)

            @pl.when(j < NCHUNK)
            def _b():
                off = pl.multiple_of(j * C, C)
                pltpu.async_copy(
                    x_sh.at[idx0_v.at[pl.ds(off, C)]], q0.at[u], sems[u])
                pltpu.async_copy(
                    x_sh.at[idx1_v.at[pl.ds(off, C)]], q1.at[u], sems[u])

            u1 = (u - 1) % NSLOT

            @pl.when(jnp.logical_and(j >= 1, j <= NCHUNK))
            def _c():
                off = pl.multiple_of((j - 1) * C, C)
                _wait_gather(u1)
                q0s = q0.at[u1]
                q1s = q1.at[u1]
                rfs = rf.at[u1]

                def widen_rows(p, _):
                    for rr in range(2):
                        r = p * 2 + rr
                        for k in range(DW // 16):
                            w0 = q0s[r, pl.ds(16 * k, 16)]
                            w1 = q1s[r, pl.ds(16 * k, 16)]
                            lo = (
                                lax.bitcast_convert_type(
                                    jnp.left_shift(w0, 16), jnp.float32)
                                + lax.bitcast_convert_type(
                                    jnp.left_shift(w1, 16), jnp.float32))
                            hi = (
                                lax.bitcast_convert_type(
                                    jnp.bitwise_and(w0, _HIMASK), jnp.float32)
                                + lax.bitcast_convert_type(
                                    jnp.bitwise_and(w1, _HIMASK), jnp.float32))
                            rfs[r, pl.ds(32 * k, 16)] = lo
                            rfs[r, pl.ds(32 * k + 16, 16)] = hi
                    return 0

                lax.fori_loop(0, C // 2, widen_rows, 0)
                pltpu.async_copy(
                    rfs, out_hbm.at[pl.ds(base + off, C)], sems[u1])
        return 0

    n_groups = (NCHUNK + 2 * NSLOT - 1) // NSLOT + 1
    lax.fori_loop(0, n_groups, group, 0)


def kernel(x, edge_index):
    ei = edge_index.astype(jnp.int32).reshape(-1)
    xp = x.astype(jnp.bfloat16)[:, _PERM]
    xq = lax.bitcast_convert_type(
        xp.reshape(N_NODES, DW, 2), jnp.int32)
    mesh = plsc.VectorSubcoreMesh(core_axis_name="c", subcore_axis_name="s")
    run = pl.kernel(
        _body,
        mesh=mesh,
        compiler_params=pltpu.CompilerParams(use_tc_tiling_on_sc=False),
        out_type=jax.ShapeDtypeStruct((E, D), jnp.float32),
        scratch_types=[
            pltpu.VMEM_SHARED((N_NODES, DW), jnp.int32),
            pltpu.VMEM((EPW,), jnp.int32),
            pltpu.VMEM((EPW,), jnp.int32),
            pltpu.VMEM((NSLOT, C, DW), jnp.int32),
            pltpu.VMEM((NSLOT, C, DW), jnp.int32),
            pltpu.VMEM((NSLOT, C, D), jnp.float32),
            pltpu.SemaphoreType.DMA,
            pltpu.SemaphoreType.DMA,
            pltpu.SemaphoreType.DMA,
            pltpu.SemaphoreType.DMA,
        ],
    )
    return run(xq, ei)


# speedup vs baseline: 1.3502x; 1.3502x over previous
import numpy as np

import jax
import jax.numpy as jnp
from jax import lax
from jax.experimental import pallas as pl
from jax.experimental.pallas import tpu as pltpu
from jax.experimental.pallas import tpu_sc as plsc

NC = 2
NS = 16
NW = NC * NS

E = 320000
D = 128
N_NODES = 10000

EPW = E // NW
C = 80
NCHUNK = EPW // C
NSLOT = 4

_PERM = np.array([32 * (m // 32) + (m % 32 % 2) * 16 + (m % 32) // 2
                  for m in range(D)], dtype=np.int32)
_HIMASK = np.int32(-65536)


def _body(xb_hbm, ei_hbm, out_hbm, x_sh, idx0_v, idx1_v, bacc, rf,
          sem0, sem1, sem2, sem3):
    sems = [sem0, sem1, sem2, sem3]
    sid = lax.axis_index("s")
    wid = sid * NC + lax.axis_index("c")
    base = pl.multiple_of(wid * EPW, EPW)

    @pl.when(sid < 5)
    def _stage():
        r0 = pl.multiple_of(sid * 2000, 2000)
        pltpu.sync_copy(xb_hbm.at[pl.ds(r0, 2000)], x_sh.at[pl.ds(r0, 2000)])

    pltpu.sync_copy(ei_hbm.at[pl.ds(base, EPW)], idx0_v)
    pltpu.sync_copy(ei_hbm.at[pl.ds(E + base, EPW)], idx1_v)
    plsc.subcore_barrier()

    def _wait_gather(slot):
        pltpu.make_async_copy(
            xb_hbm.at[pl.ds(0, C)], bacc.at[slot], sems[slot]).wait()

    def _wait_write(slot):
        pltpu.make_async_copy(
            rf.at[slot], out_hbm.at[pl.ds(0, C)], sems[slot]).wait()

    def group(t, _):
        for u in range(NSLOT):
            j = t * NSLOT + u

            @pl.when(jnp.logical_and(j >= NSLOT, j - NSLOT < NCHUNK))
            def _a():
                _wait_write(u)

            @pl.when(j < NCHUNK)
            def _b():
                off = pl.multiple_of(j * C, C)
                pltpu.async_copy(
                    x_sh.at[idx0_v.at[pl.ds(off, C)]], bacc.at[u], sems[u])

            u1 = (u - 1) % NSLOT

            @pl.when(jnp.logical_and(j >= 1, j <= NCHUNK))
            def _c():
                off = pl.multiple_of((j - 1) * C, C)
                _wait_gather(u1)
                pltpu.async_copy(
                    x_sh.at[idx1_v.at[pl.ds(off, C)]], bacc.at[u1],
                    sems[u1], add=True)

            u2 = (u - 2) % NSLOT

            @pl.when(jnp.logical_and(j >= 2, j <= NCHUNK + 1))
            def _d():
                off = pl.multiple_of((j - 2) * C, C)
                _wait_gather(u2)
                bslot = bacc.at[u2]
                rfs = rf.at[u2]

                def widen_rows(p, _):
                    for rr in range(2):
                        r = p * 2 + rr
                        for k in range(D // 32):
                            v = bslot[r, pl.ds(32 * k, 32)]
                            lo, hi = plsc.unpack(
                                v, format=plsc.PackFormat.INTERLEAVED)
                            rfs[r, pl.ds(32 * k, 16)] = lo
                            rfs[r, pl.ds(32 * k + 16, 16)] = hi
                    return 0

                lax.fori_loop(0, C // 2, widen_rows, 0)
                pltpu.async_copy(
                    rfs, out_hbm.at[pl.ds(base + off, C)], sems[u2])
        return 0

    n_groups = (NCHUNK + 2 * NSLOT - 1) // NSLOT + 1
    lax.fori_loop(0, n_groups, group, 0)


def kernel(x, edge_index):
    ei = edge_index.astype(jnp.int32).reshape(-1)
    xb = x.astype(jnp.bfloat16)[:, _PERM]
    mesh = plsc.VectorSubcoreMesh(core_axis_name="c", subcore_axis_name="s")
    run = pl.kernel(
        _body,
        mesh=mesh,
        compiler_params=pltpu.CompilerParams(use_tc_tiling_on_sc=False, needs_layout_passes=False),
        out_type=jax.ShapeDtypeStruct((E, D), jnp.float32),
        scratch_types=[
            pltpu.VMEM_SHARED((N_NODES, D), jnp.bfloat16),
            pltpu.VMEM((EPW,), jnp.int32),
            pltpu.VMEM((EPW,), jnp.int32),
            pltpu.VMEM((NSLOT, C, D), jnp.bfloat16),
            pltpu.VMEM((NSLOT, C, D), jnp.float32),
            pltpu.SemaphoreType.DMA,
            pltpu.SemaphoreType.DMA,
            pltpu.SemaphoreType.DMA,
            pltpu.SemaphoreType.DMA,
        ],
    )
    return run(xb, ei)


# speedup vs baseline: 2.1981x; 1.6280x over previous
import functools

import jax
import jax.numpy as jnp
from jax import lax
from jax.experimental import pallas as pl
from jax.experimental.pallas import tpu as pltpu
from jax.experimental.pallas import tpu_sc as plsc

NC = 2
NS = 16
NW = NC * NS

E = 320000
D = 128
N_NODES = 10000

EPW = E // NW
C = 40
NCHUNK = EPW // C


NSLOT = 4


def _body(x_hbm, ei_hbm, out_hbm, x_sh, idx0_v, idx1_v, rows,
          sem0, sem1, sem2, sem3):
    sems = [sem0, sem1, sem2, sem3]
    sid = lax.axis_index("s")
    wid = sid * NC + lax.axis_index("c")
    base = pl.multiple_of(wid * EPW, EPW)

    @pl.when(sid < 5)
    def _stage():
        r0 = pl.multiple_of(sid * 2000, 2000)
        pltpu.sync_copy(x_hbm.at[pl.ds(r0, 2000)], x_sh.at[pl.ds(r0, 2000)])

    pltpu.sync_copy(ei_hbm.at[pl.ds(base, EPW)], idx0_v)
    pltpu.sync_copy(ei_hbm.at[pl.ds(E + base, EPW)], idx1_v)
    plsc.subcore_barrier()

    def _wait(slot, dst_is_hbm):
        if dst_is_hbm:
            pltpu.make_async_copy(
                rows.at[slot], out_hbm.at[pl.ds(0, C)], sems[slot]).wait()
        else:
            pltpu.make_async_copy(
                x_hbm.at[pl.ds(0, C)], rows.at[slot], sems[slot]).wait()

    def group(t, _):
        for u in range(NSLOT):
            j = t * NSLOT + u

            @pl.when(jnp.logical_and(j >= NSLOT, j - NSLOT < NCHUNK))
            def _a():
                _wait(u, dst_is_hbm=True)

            g0_src = x_sh

            @pl.when(j < NCHUNK)
            def _b():
                off = pl.multiple_of(j * C, C)
                pltpu.async_copy(
                    g0_src.at[idx0_v.at[pl.ds(off, C)]], rows.at[u], sems[u])

            u1 = (u - 1) % NSLOT

            @pl.when(jnp.logical_and(j >= 1, j <= NCHUNK))
            def _c():
                off = pl.multiple_of((j - 1) * C, C)
                _wait(u1, dst_is_hbm=False)
                pltpu.async_copy(
                    x_sh.at[idx1_v.at[pl.ds(off, C)]], rows.at[u1],
                    sems[u1], add=True)

            u2 = (u - 2) % NSLOT

            @pl.when(jnp.logical_and(j >= 2, j <= NCHUNK + 1))
            def _d():
                off = pl.multiple_of((j - 2) * C, C)
                _wait(u2, dst_is_hbm=False)
                pltpu.async_copy(
                    rows.at[u2], out_hbm.at[pl.ds(base + off, C)], sems[u2])
        return 0

    n_groups = (NCHUNK + NSLOT + NSLOT - 1) // NSLOT + 1
    lax.fori_loop(0, n_groups, group, 0)


def kernel(x, edge_index):
    ei = edge_index.astype(jnp.int32).reshape(-1)
    mesh = plsc.VectorSubcoreMesh(core_axis_name="c", subcore_axis_name="s")
    run = pl.kernel(
        _body,
        mesh=mesh,
        compiler_params=pltpu.CompilerParams(use_tc_tiling_on_sc=False),
        out_type=jax.ShapeDtypeStruct((E, D), jnp.float32),
        scratch_types=[
            pltpu.VMEM_SHARED((N_NODES, D), jnp.float32),
            pltpu.VMEM((EPW,), jnp.int32),
            pltpu.VMEM((EPW,), jnp.int32),
            pltpu.VMEM((NSLOT, C, D), jnp.float32),
            pltpu.SemaphoreType.DMA,
            pltpu.SemaphoreType.DMA,
            pltpu.SemaphoreType.DMA,
            pltpu.SemaphoreType.DMA,
        ],
    )
    return run(x, ei)
